# baseline (speedup 1.0000x reference)
import functools

import jax
import jax.numpy as jnp
from jax.experimental import pallas as pl
from jax.experimental.pallas import tpu as pltpu

N_NODES = 10000
N_EDGES = 320000
D = 100

_BLK = 512


def _eproj_body(ea_ref, w_ref, o_ref):
    o_ref[...] = jax.lax.dot_general(
        ea_ref[...], w_ref[...], (((1,), (0,)), ((), ())),
        preferred_element_type=jnp.float32)


def _eproj(edge_attr, w_t):
    e, d = edge_attr.shape
    dout = w_t.shape[1]
    grid = (e // _BLK,)
    return pl.pallas_call(
        _eproj_body,
        grid=grid,
        in_specs=[
            pl.BlockSpec((_BLK, d), lambda i: (i, 0)),
            pl.BlockSpec((d, dout), lambda i: (0, 0)),
        ],
        out_specs=pl.BlockSpec((_BLK, dout), lambda i: (i, 0)),
        out_shape=jax.ShapeDtypeStruct((e, dout), jnp.float32),
    )(edge_attr, w_t)


def _triple_conv(x, src, dst, edge_attr, lin_W, lin_b, nn_fn):
    wi = lin_W[:, 0:D]
    we = lin_W[:, D:2 * D]
    wj = lin_W[:, 2 * D:3 * D]
    xi_proj = x @ wi.T + lin_b
    xj_proj = x @ wj.T
    e_proj = _eproj(edge_attr, we.T)
    m = jax.nn.relu(xi_proj[dst] + e_proj + xj_proj[src])
    agg = jax.ops.segment_sum(m, dst, num_segments=x.shape[0])
    return nn_fn(agg + x)


def kernel(node_features_0, node_features_1, edge_features_0, edge_features_1,
           lin1_W, lin1_b, mlp_W1, mlp_b1, mlp_W2, mlp_b2,
           lin2_W, lin2_b, mlp2_W1, mlp2_b1, mlp2_W2, mlp2_b2,
           ntn_W, ntn_V, ntn_b, rule_table, attn_W, gate_W, gate_b,
           fc1_W, fc1_b, fc2_W, fc2_b, fc3_W, fc3_b,
           edge_indices_0, edge_indices_1, rules, ori_lengths):
    def mlp1(h):
        return jax.nn.relu(h @ mlp_W1.T + mlp_b1) @ mlp_W2.T + mlp_b2

    def mlp2f(h):
        return jax.nn.relu(h @ mlp2_W1.T + mlp2_b1) @ mlp2_W2.T + mlp2_b2

    def conv_pass(x, ei, ea):
        src, dst = ei[0], ei[1]
        x = jax.nn.relu(_triple_conv(x, src, dst, ea, lin1_W, lin1_b, mlp1))
        x = jax.nn.relu(_triple_conv(x, src, dst, ea, lin2_W, lin2_b, mlp2f))
        return x

    f1 = conv_pass(node_features_0, edge_indices_0, edge_features_0)
    f2 = conv_pass(node_features_1, edge_indices_1, edge_features_1)
    g1 = f1.sum(axis=0)
    g2 = f2.sum(axis=0)
    bil = jnp.einsum('i,kij,j->k', g1, ntn_W, g2)
    graph_vector = jnp.tanh(bil + ntn_V @ jnp.concatenate([g1, g2]) + ntn_b)
    RULE_LEN = rules.shape[1]
    emb = rule_table[rules]
    mask = (jnp.arange(RULE_LEN)[None, :] < ori_lengths[:, None]).astype(jnp.float32)
    denom = jnp.maximum(ori_lengths, 1).astype(jnp.float32)[:, None]
    rules_embedding = (emb * mask[..., None]).sum(axis=1) / denom
    scores = rules_embedding @ (attn_W @ graph_vector)
    attention_weight = jax.nn.softmax(scores)
    rules_fusion = attention_weight @ rules_embedding
    gate = jax.nn.sigmoid(gate_W @ jnp.concatenate([graph_vector, rules_fusion]) + gate_b)
    final_vector = gate * graph_vector + (1.0 - gate) * rules_fusion
    x = jax.nn.relu(fc1_W @ final_vector + fc1_b)
    x = jax.nn.relu(fc2_W @ x + fc2_b)
    x = fc3_W @ x + fc3_b
    return (jnp.abs(x), attention_weight)


# speedup vs baseline: 2.6839x; 2.6839x over previous
import functools

import jax
import jax.numpy as jnp
from jax.experimental import pallas as pl
from jax.experimental.pallas import tpu as pltpu
from jax.experimental.pallas import tpu_sc as plsc

N_NODES = 10000
N_EDGES = 320000
D = 100
DIM = 64

DP = 128
NODE_PAD = 10240
DUMMY = NODE_PAD - 1
N_SUBCORES = 16
CHUNK = 112
CHUNKS_PER_TILE = -(-N_EDGES // (N_SUBCORES * CHUNK))
E_PAD = N_SUBCORES * CHUNK * CHUNKS_PER_TILE
SPMEM_ROWS = 10112
SCAT_DUMMY = N_NODES
ROWS_PER_TILE = SPMEM_ROWS // N_SUBCORES
_COPY_CHUNKS = [(0, 112), (112, 112), (224, 112), (336, 112),
                (448, 112), (560, 72)]

_f32 = jnp.float32


def _dot(a, b):
    return jax.lax.dot_general(a, b, (((1,), (0,)), ((), ())),
                               preferred_element_type=_f32)


def _proj_body(x_ref, wi_ref, wj_ref, b_ref, xi_ref, xj_ref):
    x = x_ref[...]
    xi_ref[...] = _dot(x, wi_ref[...]) + b_ref[0:1, :]
    xj_ref[...] = _dot(x, wj_ref[...])


def _proj(x_all, wi_t, wj_t, b_pad):
    n = x_all.shape[0]
    blk = 256
    return pl.pallas_call(
        _proj_body,
        grid=(n // blk,),
        in_specs=[
            pl.BlockSpec((blk, DP), lambda i: (i, 0)),
            pl.BlockSpec((DP, DP), lambda i: (0, 0)),
            pl.BlockSpec((DP, DP), lambda i: (0, 0)),
            pl.BlockSpec((8, DP), lambda i: (0, 0)),
        ],
        out_specs=[
            pl.BlockSpec((blk, DP), lambda i: (i, 0)),
            pl.BlockSpec((blk, DP), lambda i: (i, 0)),
        ],
        out_shape=[
            jax.ShapeDtypeStruct((n, DP), _f32),
            jax.ShapeDtypeStruct((n, DP), _f32),
        ],
    )(x_all, wi_t, wj_t, b_pad)


def _edge_body(ea_ref, w1_ref, w2_ref, e1_ref, e2_ref):
    ea = ea_ref[...]
    e1_ref[...] = _dot(ea, w1_ref[...])
    e2_ref[...] = _dot(ea, w2_ref[...])


def _edge(edge_attr, we1_t, we2_t):
    blk = 512
    nblk_in = N_EDGES // blk
    grid = (E_PAD // blk,)
    return pl.pallas_call(
        _edge_body,
        grid=grid,
        in_specs=[
            pl.BlockSpec((blk, D), lambda i: (jnp.minimum(i, nblk_in - 1), 0)),
            pl.BlockSpec((D, DP), lambda i: (0, 0)),
            pl.BlockSpec((D, DP), lambda i: (0, 0)),
        ],
        out_specs=[
            pl.BlockSpec((blk, DP), lambda i: (i, 0)),
            pl.BlockSpec((blk, DP), lambda i: (i, 0)),
        ],
        out_shape=[
            jax.ShapeDtypeStruct((E_PAD, DP), _f32),
            jax.ShapeDtypeStruct((E_PAD, DP), _f32),
        ],
    )(edge_attr, we1_t, we2_t)


def _post1_body(agg_ref, x_ref, mw1_ref, mb1_ref, mw2_ref, mb2_ref,
                wi2_ref, wj2_ref, b2_ref, x1_ref, xi2_ref, xj2_ref):
    h = agg_ref[...] + x_ref[...]
    t = jax.nn.relu(_dot(h, mw1_ref[...]) + mb1_ref[0:1, :])
    x1 = jax.nn.relu(_dot(t, mw2_ref[...]) + mb2_ref[0:1, :])
    x1_ref[...] = x1
    xi2_ref[...] = _dot(x1, wi2_ref[...]) + b2_ref[0:1, :]
    xj2_ref[...] = _dot(x1, wj2_ref[...])


def _post1(agg_all, x_all, mw1_t, mb1, mw2_t, mb2, wi2_t, wj2_t, b2):
    n = x_all.shape[0]
    blk = 256
    wspec = pl.BlockSpec((DP, DP), lambda i: (0, 0))
    bspec = pl.BlockSpec((8, DP), lambda i: (0, 0))
    rspec = pl.BlockSpec((blk, DP), lambda i: (i, 0))
    return pl.pallas_call(
        _post1_body,
        grid=(n // blk,),
        in_specs=[rspec, rspec, wspec, bspec, wspec, bspec, wspec, wspec,
                  bspec],
        out_specs=[rspec, rspec, rspec],
        out_shape=[jax.ShapeDtypeStruct((n, DP), _f32)] * 3,
    )(agg_all, x_all, mw1_t, mb1, mw2_t, mb2, wi2_t, wj2_t, b2)


def _post2_body(agg_ref, x1_ref, mw1_ref, mb1_ref, mw2_ref, mb2_ref,
                out_ref):
    i = pl.program_id(0)
    blk = agg_ref.shape[0]
    blocks_per_graph = NODE_PAD // blk
    h = agg_ref[...] + x1_ref[...]
    t = jax.nn.relu(_dot(h, mw1_ref[...]) + mb1_ref[0:1, :])
    x2 = jax.nn.relu(_dot(t, mw2_ref[...]) + mb2_ref[0:1, :])
    local_row = (i % blocks_per_graph) * blk + jax.lax.broadcasted_iota(
        jnp.int32, (blk, 1), 0)
    x2 = jnp.where(local_row < N_NODES, x2, 0.0)
    part = x2.reshape(blk // 8, 8, DP).sum(axis=0)

    @pl.when(i % blocks_per_graph == 0)
    def _():
        out_ref[...] = jnp.zeros_like(out_ref)

    out_ref[...] += part


def _post2(agg_all, x1_all, mw1_t, mb1, mw2_t, mb2):
    n = x1_all.shape[0]
    blk = 256
    wspec = pl.BlockSpec((DP, DP), lambda i: (0, 0))
    bspec = pl.BlockSpec((8, DP), lambda i: (0, 0))
    rspec = pl.BlockSpec((blk, DP), lambda i: (i, 0))
    blocks_per_graph = NODE_PAD // blk
    return pl.pallas_call(
        _post2_body,
        grid=(n // blk,),
        in_specs=[rspec, rspec, wspec, bspec, wspec, bspec],
        out_specs=pl.BlockSpec((8, DP), lambda i: (i // blocks_per_graph, 0)),
        out_shape=jax.ShapeDtypeStruct((16, DP), _f32),
    )(agg_all, x1_all, mw1_t, mb1, mw2_t, mb2)


def _sc_conv(xi_all, xj_all, e0, e1, sdx, ddx, ddr):
    mesh = plsc.VectorSubcoreMesh(core_axis_name="c", subcore_axis_name="s")

    @functools.partial(
        pl.kernel, mesh=mesh,
        out_type=jax.ShapeDtypeStruct((2 * NODE_PAD, DP), _f32),
        scratch_types=[
            pltpu.VMEM_SHARED((SPMEM_ROWS, DP), _f32),
            pltpu.VMEM((CHUNK,), jnp.int32),
            pltpu.VMEM((CHUNK,), jnp.int32),
            pltpu.VMEM((CHUNK,), jnp.int32),
            pltpu.VMEM((CHUNK, DP), _f32),
            pltpu.VMEM((CHUNK, DP), _f32),
            pltpu.VMEM((CHUNK, DP), _f32),
            pltpu.SemaphoreType.DMA,
            pltpu.SemaphoreType.DMA,
        ],
    )
    def k(xi_h, xj_h, e0_h, e1_h, sdx_h, ddx_h, ddr_h, out_h,
          agg_sh, idx_s, idx_d, idx_r, rows_i, rows_j, rows_e,
          sem_a, sem_b):
        c = jax.lax.axis_index("c")
        s = jax.lax.axis_index("s")
        row0 = s * ROWS_PER_TILE

        def zrow(i, _):
            for j in range(DP // 16):
                rows_e[i, pl.ds(j * 16, 16)] = jnp.zeros((16,), _f32)
            return 0
        jax.lax.fori_loop(0, CHUNK, zrow, 0)
        for off, sz in _COPY_CHUNKS:
            pltpu.sync_copy(rows_e.at[pl.ds(0, sz)],
                            agg_sh.at[pl.ds(row0 + off, sz)])
        plsc.subcore_barrier()

        tile_edges = CHUNKS_PER_TILE * CHUNK
        ebase = c * E_PAD + s * tile_edges
        lbase0 = s * tile_edges

        def chunk_body(kk, _):
            base = ebase + kk * CHUNK
            pltpu.sync_copy(sdx_h.at[pl.ds(base, CHUNK)], idx_s)
            pltpu.sync_copy(ddx_h.at[pl.ds(base, CHUNK)], idx_d)
            pltpu.sync_copy(ddr_h.at[pl.ds(base, CHUNK)], idx_r)
            cp_i = pltpu.async_copy(xi_h.at[idx_d], rows_i, sem_a)
            cp_j = pltpu.async_copy(xj_h.at[idx_s], rows_j, sem_b)
            lbase = lbase0 + kk * CHUNK

            @pl.when(c == 0)
            def _():
                pltpu.sync_copy(e0_h.at[pl.ds(lbase, CHUNK)], rows_e)

            @pl.when(c == 1)
            def _():
                pltpu.sync_copy(e1_h.at[pl.ds(lbase, CHUNK)], rows_e)

            cp_i.wait()
            cp_j.wait()

            def row(i, _):
                for j in range(DP // 16):
                    sl = pl.ds(j * 16, 16)
                    v = rows_e[i, sl] + rows_i[i, sl] + rows_j[i, sl]
                    rows_e[i, sl] = jnp.maximum(v, 0.0)
                return 0
            jax.lax.fori_loop(0, CHUNK, row, 0)
            pltpu.sync_copy(rows_e, agg_sh.at[idx_r], add=True)
            return 0

        jax.lax.fori_loop(0, CHUNKS_PER_TILE, chunk_body, 0)
        plsc.subcore_barrier()

        obase = c * NODE_PAD + row0
        for off, sz in _COPY_CHUNKS:
            pltpu.sync_copy(agg_sh.at[pl.ds(row0 + off, sz)],
                            rows_e.at[pl.ds(0, sz)])
            pltpu.sync_copy(rows_e.at[pl.ds(0, sz)],
                            out_h.at[pl.ds(obase + off, sz)])

    return k(xi_all, xj_all, e0, e1, sdx, ddx, ddr)


def _padw(w, r, c):
    return jnp.zeros((r, c), _f32).at[:w.shape[0], :w.shape[1]].set(w)


def _padb(b):
    return jnp.zeros((8, DP), _f32).at[0, :b.shape[0]].set(b)


def _padidx(a, fill):
    return jnp.full((E_PAD,), fill, jnp.int32).at[:N_EDGES].set(a)


def kernel(node_features_0, node_features_1, edge_features_0, edge_features_1,
           lin1_W, lin1_b, mlp_W1, mlp_b1, mlp_W2, mlp_b2,
           lin2_W, lin2_b, mlp2_W1, mlp2_b1, mlp2_W2, mlp2_b2,
           ntn_W, ntn_V, ntn_b, rule_table, attn_W, gate_W, gate_b,
           fc1_W, fc1_b, fc2_W, fc2_b, fc3_W, fc3_b,
           edge_indices_0, edge_indices_1, rules, ori_lengths):
    x_all = (jnp.zeros((2 * NODE_PAD, DP), _f32)
             .at[:N_NODES, :D].set(node_features_0)
             .at[NODE_PAD:NODE_PAD + N_NODES, :D].set(node_features_1))

    wi1_t = _padw(lin1_W[:, 0:D].T, DP, DP)
    we1_t = _padw(lin1_W[:, D:2 * D].T, D, DP)
    wj1_t = _padw(lin1_W[:, 2 * D:3 * D].T, DP, DP)
    b1 = _padb(lin1_b)
    wi2_t = _padw(lin2_W[:, 0:D].T, DP, DP)
    we2_t = _padw(lin2_W[:, D:2 * D].T, D, DP)
    wj2_t = _padw(lin2_W[:, 2 * D:3 * D].T, DP, DP)
    b2 = _padb(lin2_b)
    mw1_t = _padw(mlp_W1.T, DP, DP)
    mb1 = _padb(mlp_b1)
    mw2_t = _padw(mlp_W2.T, DP, DP)
    mb2 = _padb(mlp_b2)
    m2w1_t = _padw(mlp2_W1.T, DP, DP)
    m2b1 = _padb(mlp2_b1)
    m2w2_t = _padw(mlp2_W2.T, DP, DP)
    m2b2 = _padb(mlp2_b2)

    src0, dst0 = edge_indices_0[0], edge_indices_0[1]
    src1, dst1 = edge_indices_1[0], edge_indices_1[1]
    sdx = jnp.concatenate([_padidx(src0, DUMMY),
                           _padidx(src1, DUMMY) + NODE_PAD])
    ddx = jnp.concatenate([_padidx(dst0, DUMMY),
                           _padidx(dst1, DUMMY) + NODE_PAD])
    ddr = jnp.concatenate([_padidx(dst0, SCAT_DUMMY),
                           _padidx(dst1, SCAT_DUMMY)])

    xi1_all, xj1_all = _proj(x_all, wi1_t, wj1_t, b1)
    e1_0, e2_0 = _edge(edge_features_0, we1_t, we2_t)
    e1_1, e2_1 = _edge(edge_features_1, we1_t, we2_t)
    agg1 = _sc_conv(xi1_all, xj1_all, e1_0, e1_1, sdx, ddx, ddr)
    x1_all, xi2_all, xj2_all = _post1(
        agg1, x_all, mw1_t, mb1, mw2_t, mb2, wi2_t, wj2_t, b2)

    agg2 = _sc_conv(xi2_all, xj2_all, e2_0, e2_1, sdx, ddx, ddr)
    colsum = _post2(agg2, x1_all, m2w1_t, m2b1, m2w2_t, m2b2)
    g1 = colsum[0:8].sum(axis=0)[:DIM]
    g2 = colsum[8:16].sum(axis=0)[:DIM]

    bil = jnp.einsum('i,kij,j->k', g1, ntn_W, g2)
    graph_vector = jnp.tanh(bil + ntn_V @ jnp.concatenate([g1, g2]) + ntn_b)
    rule_len = rules.shape[1]
    emb = rule_table[rules]
    mask = (jnp.arange(rule_len)[None, :] < ori_lengths[:, None]).astype(_f32)
    denom = jnp.maximum(ori_lengths, 1).astype(_f32)[:, None]
    rules_embedding = (emb * mask[..., None]).sum(axis=1) / denom
    scores = rules_embedding @ (attn_W @ graph_vector)
    attention_weight = jax.nn.softmax(scores)
    rules_fusion = attention_weight @ rules_embedding
    gate = jax.nn.sigmoid(
        gate_W @ jnp.concatenate([graph_vector, rules_fusion]) + gate_b)
    final_vector = gate * graph_vector + (1.0 - gate) * rules_fusion
    x = jax.nn.relu(fc1_W @ final_vector + fc1_b)
    x = jax.nn.relu(fc2_W @ x + fc2_b)
    x = fc3_W @ x + fc3_b
    return (jnp.abs(x), attention_weight)


# speedup vs baseline: 3.3931x; 1.2642x over previous
import functools

import jax
import jax.numpy as jnp
from jax.experimental import pallas as pl
from jax.experimental.pallas import tpu as pltpu
from jax.experimental.pallas import tpu_sc as plsc

N_NODES = 10000
N_EDGES = 320000
D = 100
DIM = 64

DP = 128
NODE_PAD = 10240
DUMMY = NODE_PAD - 1
N_SUBCORES = 16
CHUNK = 56
CHUNKS_PER_TILE = 360
E_PAD = N_SUBCORES * CHUNK * CHUNKS_PER_TILE
SPMEM_ROWS = 10112
SCAT_DUMMY = N_NODES
ROWS_PER_TILE = SPMEM_ROWS // N_SUBCORES
_COPY_CHUNKS = [(i * CHUNK, CHUNK) for i in range(11)] + [(616, 16)]

_f32 = jnp.float32


def _dot(a, b):
    return jax.lax.dot_general(a, b, (((1,), (0,)), ((), ())),
                               preferred_element_type=_f32)


def _proj_body(x_ref, wi_ref, wj_ref, b_ref, xi_ref, xj_ref):
    x = x_ref[...]
    xi_ref[...] = _dot(x, wi_ref[...]) + b_ref[0:1, :]
    xj_ref[...] = _dot(x, wj_ref[...])


def _proj(x_all, wi_t, wj_t, b_pad):
    n = x_all.shape[0]
    blk = 256
    return pl.pallas_call(
        _proj_body,
        grid=(n // blk,),
        in_specs=[
            pl.BlockSpec((blk, DP), lambda i: (i, 0)),
            pl.BlockSpec((DP, DP), lambda i: (0, 0)),
            pl.BlockSpec((DP, DP), lambda i: (0, 0)),
            pl.BlockSpec((8, DP), lambda i: (0, 0)),
        ],
        out_specs=[
            pl.BlockSpec((blk, DP), lambda i: (i, 0)),
            pl.BlockSpec((blk, DP), lambda i: (i, 0)),
        ],
        out_shape=[
            jax.ShapeDtypeStruct((n, DP), _f32),
            jax.ShapeDtypeStruct((n, DP), _f32),
        ],
    )(x_all, wi_t, wj_t, b_pad)


def _edge_body(ea_ref, w1_ref, w2_ref, e1_ref, e2_ref):
    ea = ea_ref[...]
    e1_ref[...] = _dot(ea, w1_ref[...])
    e2_ref[...] = _dot(ea, w2_ref[...])


def _edge(edge_attr, we1_t, we2_t):
    blk = 512
    nblk_in = N_EDGES // blk
    grid = (E_PAD // blk,)
    return pl.pallas_call(
        _edge_body,
        grid=grid,
        in_specs=[
            pl.BlockSpec((blk, D), lambda i: (jnp.minimum(i, nblk_in - 1), 0)),
            pl.BlockSpec((D, DP), lambda i: (0, 0)),
            pl.BlockSpec((D, DP), lambda i: (0, 0)),
        ],
        out_specs=[
            pl.BlockSpec((blk, DP), lambda i: (i, 0)),
            pl.BlockSpec((blk, DP), lambda i: (i, 0)),
        ],
        out_shape=[
            jax.ShapeDtypeStruct((E_PAD, DP), _f32),
            jax.ShapeDtypeStruct((E_PAD, DP), _f32),
        ],
    )(edge_attr, we1_t, we2_t)


def _post1_body(agg_ref, x_ref, mw1_ref, mb1_ref, mw2_ref, mb2_ref,
                wi2_ref, wj2_ref, b2_ref, x1_ref, xi2_ref, xj2_ref):
    h = agg_ref[...] + x_ref[...]
    t = jax.nn.relu(_dot(h, mw1_ref[...]) + mb1_ref[0:1, :])
    x1 = jax.nn.relu(_dot(t, mw2_ref[...]) + mb2_ref[0:1, :])
    x1_ref[...] = x1
    xi2_ref[...] = _dot(x1, wi2_ref[...]) + b2_ref[0:1, :]
    xj2_ref[...] = _dot(x1, wj2_ref[...])


def _post1(agg_all, x_all, mw1_t, mb1, mw2_t, mb2, wi2_t, wj2_t, b2):
    n = x_all.shape[0]
    blk = 256
    wspec = pl.BlockSpec((DP, DP), lambda i: (0, 0))
    bspec = pl.BlockSpec((8, DP), lambda i: (0, 0))
    rspec = pl.BlockSpec((blk, DP), lambda i: (i, 0))
    return pl.pallas_call(
        _post1_body,
        grid=(n // blk,),
        in_specs=[rspec, rspec, wspec, bspec, wspec, bspec, wspec, wspec,
                  bspec],
        out_specs=[rspec, rspec, rspec],
        out_shape=[jax.ShapeDtypeStruct((n, DP), _f32)] * 3,
    )(agg_all, x_all, mw1_t, mb1, mw2_t, mb2, wi2_t, wj2_t, b2)


def _post2_body(agg_ref, x1_ref, mw1_ref, mb1_ref, mw2_ref, mb2_ref,
                out_ref):
    i = pl.program_id(0)
    blk = agg_ref.shape[0]
    blocks_per_graph = NODE_PAD // blk
    h = agg_ref[...] + x1_ref[...]
    t = jax.nn.relu(_dot(h, mw1_ref[...]) + mb1_ref[0:1, :])
    x2 = jax.nn.relu(_dot(t, mw2_ref[...]) + mb2_ref[0:1, :])
    local_row = (i % blocks_per_graph) * blk + jax.lax.broadcasted_iota(
        jnp.int32, (blk, 1), 0)
    x2 = jnp.where(local_row < N_NODES, x2, 0.0)
    part = x2.reshape(blk // 8, 8, DP).sum(axis=0)

    @pl.when(i % blocks_per_graph == 0)
    def _():
        out_ref[...] = jnp.zeros_like(out_ref)

    out_ref[...] += part


def _post2(agg_all, x1_all, mw1_t, mb1, mw2_t, mb2):
    n = x1_all.shape[0]
    blk = 256
    wspec = pl.BlockSpec((DP, DP), lambda i: (0, 0))
    bspec = pl.BlockSpec((8, DP), lambda i: (0, 0))
    rspec = pl.BlockSpec((blk, DP), lambda i: (i, 0))
    blocks_per_graph = NODE_PAD // blk
    return pl.pallas_call(
        _post2_body,
        grid=(n // blk,),
        in_specs=[rspec, rspec, wspec, bspec, wspec, bspec],
        out_specs=pl.BlockSpec((8, DP), lambda i: (i // blocks_per_graph, 0)),
        out_shape=jax.ShapeDtypeStruct((16, DP), _f32),
    )(agg_all, x1_all, mw1_t, mb1, mw2_t, mb2)


def _sc_conv(xi_all, xj_all, e0, e1, idx_il):
    mesh = plsc.VectorSubcoreMesh(core_axis_name="c", subcore_axis_name="s")
    nct = CHUNKS_PER_TILE

    @functools.partial(
        pl.kernel, mesh=mesh,
        out_type=jax.ShapeDtypeStruct((2 * NODE_PAD, DP), _f32),
        scratch_types=[
            pltpu.VMEM_SHARED((SPMEM_ROWS, DP), _f32),
            pltpu.VMEM((3, 3, CHUNK), jnp.int32),
            pltpu.VMEM((2, CHUNK, DP), _f32),
            pltpu.VMEM((2, CHUNK, DP), _f32),
            pltpu.VMEM((2, CHUNK, DP), _f32),
        ] + [pltpu.SemaphoreType.DMA] * 11,
    )
    def k(xi_h, xj_h, e0_h, e1_h, idx_h, out_h,
          agg_sh, idxb, ri, rj, re,
          sx0, sx1, sx2, si0, si1, sj0, sj1, se0, se1, ss0, ss1):
        sem_idx = [sx0, sx1, sx2]
        sem_i = [si0, si1]
        sem_j = [sj0, sj1]
        sem_e = [se0, se1]
        sem_sc = [ss0, ss1]
        c = jax.lax.axis_index("c")
        s = jax.lax.axis_index("s")
        row0 = s * ROWS_PER_TILE
        cbase = (c * N_SUBCORES + s) * nct
        lbase0 = s * (nct * CHUNK)

        def idx_fetch(kk, islot):
            pltpu.async_copy(idx_h.at[cbase + kk], idxb.at[islot],
                             sem_idx[islot])

        def idx_wait(islot):
            pltpu.make_async_copy(idx_h.at[0], idxb.at[islot],
                                  sem_idx[islot]).wait()

        def gav_start(kk, b, islot):
            lb = lbase0 + kk * CHUNK

            @pl.when(c == 0)
            def _():
                pltpu.async_copy(e0_h.at[pl.ds(lb, CHUNK)], re.at[b],
                                 sem_e[b])

            @pl.when(c == 1)
            def _():
                pltpu.async_copy(e1_h.at[pl.ds(lb, CHUNK)], re.at[b],
                                 sem_e[b])

            pltpu.async_copy(xi_h.at[idxb.at[islot, 1]], ri.at[b], sem_i[b])
            pltpu.async_copy(xj_h.at[idxb.at[islot, 0]], rj.at[b], sem_j[b])

        def gav_wait(b, islot):
            pltpu.make_async_copy(e0_h.at[pl.ds(0, CHUNK)], re.at[b],
                                  sem_e[b]).wait()
            pltpu.make_async_copy(xi_h.at[idxb.at[islot, 1]], ri.at[b],
                                  sem_i[b]).wait()
            pltpu.make_async_copy(xj_h.at[idxb.at[islot, 0]], rj.at[b],
                                  sem_j[b]).wait()

        def scat_start(b, islot):
            pltpu.async_copy(re.at[b], agg_sh.at[idxb.at[islot, 2]],
                             sem_sc[b], add=True)

        def scat_wait(b):
            pltpu.make_async_copy(re.at[b], agg_sh.at[pl.ds(0, CHUNK)],
                                  sem_sc[b]).wait()

        def compute(b):
            def row(i, _):
                for j in range(DP // 16):
                    sl = pl.ds(j * 16, 16)
                    v = re[b, i, sl] + ri[b, i, sl] + rj[b, i, sl]
                    re[b, i, sl] = jnp.maximum(v, 0.0)
                return 0
            jax.lax.fori_loop(0, CHUNK, row, 0)

        idx_fetch(0, 0)
        idx_fetch(1, 1)

        def zrow(i, _):
            for j in range(DP // 16):
                re[0, i, pl.ds(j * 16, 16)] = jnp.zeros((16,), _f32)
            return 0
        jax.lax.fori_loop(0, CHUNK, zrow, 0)
        for off, sz in _COPY_CHUNKS:
            pltpu.sync_copy(re.at[0, pl.ds(0, sz)],
                            agg_sh.at[pl.ds(row0 + off, sz)])
        plsc.subcore_barrier()

        idx_wait(0)
        gav_start(0, 0, 0)

        T = nct // 6

        def body6(t, _):
            k0 = t * 6
            for u in range(6):
                kk = k0 + u
                b, o = u % 2, 1 - u % 2
                icur, inxt, ipre = u % 3, (u + 1) % 3, (u + 2) % 3
                gav_wait(b, icur)

                def launch():
                    idx_wait(inxt)
                    if u == 0:
                        @pl.when(t > 0)
                        def _():
                            scat_wait(o)
                    else:
                        scat_wait(o)
                    gav_start(kk + 1, o, inxt)
                if u < 5:
                    launch()
                else:
                    @pl.when(t < T - 1)
                    def _():
                        launch()

                if u < 4:
                    idx_fetch(kk + 2, ipre)
                else:
                    @pl.when(t < T - 1)
                    def _():
                        idx_fetch(kk + 2, ipre)

                compute(b)
                scat_start(b, icur)
            return 0

        jax.lax.fori_loop(0, T, body6, 0)
        scat_wait(0)
        scat_wait(1)
        plsc.subcore_barrier()

        obase = c * NODE_PAD + row0
        for off, sz in _COPY_CHUNKS:
            pltpu.sync_copy(agg_sh.at[pl.ds(row0 + off, sz)],
                            re.at[0, pl.ds(0, sz)])
            pltpu.sync_copy(re.at[0, pl.ds(0, sz)],
                            out_h.at[pl.ds(obase + off, sz)])

    return k(xi_all, xj_all, e0, e1, idx_il)


def _padw(w, r, c):
    return jnp.zeros((r, c), _f32).at[:w.shape[0], :w.shape[1]].set(w)


def _padb(b):
    return jnp.zeros((8, DP), _f32).at[0, :b.shape[0]].set(b)


def _padidx(a, fill):
    return jnp.full((E_PAD,), fill, jnp.int32).at[:N_EDGES].set(a)


def kernel(node_features_0, node_features_1, edge_features_0, edge_features_1,
           lin1_W, lin1_b, mlp_W1, mlp_b1, mlp_W2, mlp_b2,
           lin2_W, lin2_b, mlp2_W1, mlp2_b1, mlp2_W2, mlp2_b2,
           ntn_W, ntn_V, ntn_b, rule_table, attn_W, gate_W, gate_b,
           fc1_W, fc1_b, fc2_W, fc2_b, fc3_W, fc3_b,
           edge_indices_0, edge_indices_1, rules, ori_lengths):
    x_all = (jnp.zeros((2 * NODE_PAD, DP), _f32)
             .at[:N_NODES, :D].set(node_features_0)
             .at[NODE_PAD:NODE_PAD + N_NODES, :D].set(node_features_1))

    wi1_t = _padw(lin1_W[:, 0:D].T, DP, DP)
    we1_t = _padw(lin1_W[:, D:2 * D].T, D, DP)
    wj1_t = _padw(lin1_W[:, 2 * D:3 * D].T, DP, DP)
    b1 = _padb(lin1_b)
    wi2_t = _padw(lin2_W[:, 0:D].T, DP, DP)
    we2_t = _padw(lin2_W[:, D:2 * D].T, D, DP)
    wj2_t = _padw(lin2_W[:, 2 * D:3 * D].T, DP, DP)
    b2 = _padb(lin2_b)
    mw1_t = _padw(mlp_W1.T, DP, DP)
    mb1 = _padb(mlp_b1)
    mw2_t = _padw(mlp_W2.T, DP, DP)
    mb2 = _padb(mlp_b2)
    m2w1_t = _padw(mlp2_W1.T, DP, DP)
    m2b1 = _padb(mlp2_b1)
    m2w2_t = _padw(mlp2_W2.T, DP, DP)
    m2b2 = _padb(mlp2_b2)

    src0, dst0 = edge_indices_0[0], edge_indices_0[1]
    src1, dst1 = edge_indices_1[0], edge_indices_1[1]
    sdx = jnp.concatenate([_padidx(src0, DUMMY),
                           _padidx(src1, DUMMY) + NODE_PAD])
    ddx = jnp.concatenate([_padidx(dst0, DUMMY),
                           _padidx(dst1, DUMMY) + NODE_PAD])
    ddr = jnp.concatenate([_padidx(dst0, SCAT_DUMMY),
                           _padidx(dst1, SCAT_DUMMY)])
    shp = (2, N_SUBCORES, CHUNKS_PER_TILE, CHUNK)
    idx_il = jnp.stack(
        [sdx.reshape(shp), ddx.reshape(shp), ddr.reshape(shp)],
        axis=3).reshape(2 * N_SUBCORES * CHUNKS_PER_TILE, 3, CHUNK)

    xi1_all, xj1_all = _proj(x_all, wi1_t, wj1_t, b1)
    e1_0, e2_0 = _edge(edge_features_0, we1_t, we2_t)
    e1_1, e2_1 = _edge(edge_features_1, we1_t, we2_t)
    agg1 = _sc_conv(xi1_all, xj1_all, e1_0, e1_1, idx_il)
    x1_all, xi2_all, xj2_all = _post1(
        agg1, x_all, mw1_t, mb1, mw2_t, mb2, wi2_t, wj2_t, b2)

    agg2 = _sc_conv(xi2_all, xj2_all, e2_0, e2_1, idx_il)
    colsum = _post2(agg2, x1_all, m2w1_t, m2b1, m2w2_t, m2b2)
    g1 = colsum[0:8].sum(axis=0)[:DIM]
    g2 = colsum[8:16].sum(axis=0)[:DIM]

    bil = jnp.einsum('i,kij,j->k', g1, ntn_W, g2)
    graph_vector = jnp.tanh(bil + ntn_V @ jnp.concatenate([g1, g2]) + ntn_b)
    rule_len = rules.shape[1]
    emb = rule_table[rules]
    mask = (jnp.arange(rule_len)[None, :] < ori_lengths[:, None]).astype(_f32)
    denom = jnp.maximum(ori_lengths, 1).astype(_f32)[:, None]
    rules_embedding = (emb * mask[..., None]).sum(axis=1) / denom
    scores = rules_embedding @ (attn_W @ graph_vector)
    attention_weight = jax.nn.softmax(scores)
    rules_fusion = attention_weight @ rules_embedding
    gate = jax.nn.sigmoid(
        gate_W @ jnp.concatenate([graph_vector, rules_fusion]) + gate_b)
    final_vector = gate * graph_vector + (1.0 - gate) * rules_fusion
    x = jax.nn.relu(fc1_W @ final_vector + fc1_b)
    x = jax.nn.relu(fc2_W @ x + fc2_b)
    x = fc3_W @ x + fc3_b
    return (jnp.abs(x), attention_weight)
